# baseline (speedup 1.0000x reference)
import jax
import jax.numpy as jnp
from jax.experimental import pallas as pl
from jax.experimental.pallas import tpu as pltpu

N_RES = 38
NB = 4
BN = 1000


def _band_out(xb, w, b):
    cols = jax.lax.broadcasted_iota(jnp.int32, xb.shape, 1)
    in_head = cols < N_RES
    head = jnp.where(in_head, xb, -jnp.inf)
    mx = jnp.max(head, axis=1, keepdims=True)
    idx = jnp.min(jnp.where(head == mx, cols, jnp.int32(10**9)),
                  axis=1, keepdims=True)
    onehot_or_x = jnp.where(in_head, (cols == idx).astype(xb.dtype), xb)
    return jnp.dot(onehot_or_x, w, preferred_element_type=jnp.float32) + b


def _body(*refs):
    x_refs = refs[:NB]
    w_ref, b_ref, o_ref = refs[NB:]
    w = w_ref[...]
    b = b_ref[...]
    for j in range(NB):
        o_ref[j] = _band_out(x_refs[j][...], w, b)


def kernel(x, emb_table, W_feats, b_feats, W_sigma, b_sigma):
    n, d = x.shape
    n_s = emb_table.shape[1]
    rows_b = n // NB
    steps = rows_b // BN
    w_full = jnp.concatenate([emb_table, W_feats.T, W_sigma.T], axis=0)
    bias = (b_feats + b_sigma)[None, :]
    in_specs = [
        pl.BlockSpec((BN, d), lambda i, j=j: (i + j * steps, 0))
        for j in range(NB)
    ]
    in_specs.append(pl.BlockSpec((d, n_s), lambda i: (0, 0)))
    in_specs.append(pl.BlockSpec((1, n_s), lambda i: (0, 0)))
    out = pl.pallas_call(
        _body,
        grid=(steps,),
        in_specs=in_specs,
        out_specs=pl.BlockSpec((NB, BN, n_s), lambda i: (0, i, 0)),
        out_shape=jax.ShapeDtypeStruct((NB, rows_b, n_s), jnp.float32),
        compiler_params=pltpu.CompilerParams(
            dimension_semantics=("parallel",),
        ),
    )(*([x] * NB), w_full, bias)
    return out.reshape(n, n_s)


# speedup vs baseline: 1.0096x; 1.0096x over previous
import jax
import jax.numpy as jnp
from jax.experimental import pallas as pl
from jax.experimental.pallas import tpu as pltpu

N_RES = 38
BN = 4000
NBUF = 4


def _make_body(steps):
    def _body(x_hbm, w_ref, b_ref, o_ref, xs, sems):
        i = pl.program_id(0)

        def start(blk, slot):
            pltpu.make_async_copy(
                x_hbm.at[pl.ds(blk * BN, BN), :], xs.at[slot], sems.at[slot]
            ).start()

        @pl.when(i == 0)
        def _prologue():
            for k in range(min(NBUF - 1, steps)):
                start(k, k)

        nxt = i + NBUF - 1

        @pl.when(nxt < steps)
        def _prefetch():
            start(nxt, nxt % NBUF)

        slot = i % NBUF
        pltpu.make_async_copy(
            x_hbm.at[pl.ds(i * BN, BN), :], xs.at[slot], sems.at[slot]
        ).wait()

        xb = xs[slot]
        cols = jax.lax.broadcasted_iota(jnp.int32, xb.shape, 1)
        in_head = cols < N_RES
        head = jnp.where(in_head, xb, -jnp.inf)
        mx = jnp.max(head, axis=1, keepdims=True)
        idx = jnp.min(jnp.where(head == mx, cols, jnp.int32(10**9)),
                      axis=1, keepdims=True)
        onehot_or_x = jnp.where(in_head, (cols == idx).astype(xb.dtype), xb)
        o_ref[...] = (
            jnp.dot(onehot_or_x, w_ref[...], preferred_element_type=jnp.float32)
            + b_ref[...]
        )

    return _body


def kernel(x, emb_table, W_feats, b_feats, W_sigma, b_sigma):
    n, d = x.shape
    n_s = emb_table.shape[1]
    w_full = jnp.concatenate([emb_table, W_feats.T, W_sigma.T], axis=0)
    bias = (b_feats + b_sigma)[None, :]
    return pl.pallas_call(
        _make_body(n // BN),
        grid=(n // BN,),
        in_specs=[
            pl.BlockSpec(memory_space=pl.ANY),
            pl.BlockSpec((d, n_s), lambda i: (0, 0)),
            pl.BlockSpec((1, n_s), lambda i: (0, 0)),
        ],
        out_specs=pl.BlockSpec((BN, n_s), lambda i: (i, 0)),
        out_shape=jax.ShapeDtypeStruct((n, n_s), jnp.float32),
        scratch_shapes=[
            pltpu.VMEM((NBUF, BN, d), jnp.float32),
            pltpu.SemaphoreType.DMA((NBUF,)),
        ],
        compiler_params=pltpu.CompilerParams(
            dimension_semantics=("arbitrary",),
        ),
    )(x, w_full, bias)
